# baseline (speedup 1.0000x reference)
import functools

import jax
import jax.numpy as jnp
from jax import lax
from jax.experimental import pallas as pl
from jax.experimental.pallas import tpu as pltpu
from jax.experimental.pallas import tpu_sc as plsc

_BATCH = 4096
_HIST = 50
_DIM = 64
_PDIM = 128
_VOCAB = 1000000
_NW = 32
_LANE = 16

_NBLK = _VOCAB // _PDIM
_BLK_BASE = _NBLK // _NW
_BLK_REM = _NBLK - _BLK_BASE * _NW

_CHUNK = 400
_ROWS_PER_W = (_BATCH * _HIST) // _NW
_NCHUNK = _ROWS_PER_W // _CHUNK
_NBUF = 2


def _build_detile():
    mesh = plsc.VectorSubcoreMesh(core_axis_name="c", subcore_axis_name="s")

    @functools.partial(
        pl.kernel,
        out_type=jax.ShapeDtypeStruct((_VOCAB, _PDIM), jnp.float32),
        mesh=mesh,
        scratch_types=[
            pltpu.VMEM((_DIM, _PDIM), jnp.float32),
            pltpu.VMEM((_DIM, _DIM), jnp.float32),
            pltpu.VMEM((_PDIM, _PDIM), jnp.float32),
        ],
        compiler_params=pltpu.CompilerParams(
            use_tc_tiling_on_sc=True, needs_layout_passes=False),
    )
    def detile_kernel(tt_hbm, tail_hbm, s_hbm, blk_v, tail_v, tr_v):
        wid = lax.axis_index("s") * 2 + lax.axis_index("c")
        start = wid * _BLK_BASE + jnp.minimum(wid, _BLK_REM)
        cnt = _BLK_BASE + jnp.where(wid < _BLK_REM, 1, 0)
        iota = lax.iota(jnp.int32, _LANE)

        def transpose_block(src, width):
            for c in range(_DIM):
                cvec = jnp.full((_LANE,), c, jnp.int32)
                for k in range(width // _LANE):
                    v = src[c, pl.ds(k * _LANE, _LANE)]
                    plsc.store_scatter(tr_v, [iota + k * _LANE, cvec], v)

        def body(j, carry):
            pltpu.sync_copy(tt_hbm.at[:, pl.ds(j * _PDIM, _PDIM)], blk_v)
            transpose_block(blk_v, _PDIM)
            pltpu.sync_copy(tr_v, s_hbm.at[pl.ds(j * _PDIM, _PDIM), :])
            return carry

        lax.fori_loop(start, start + cnt, body, 0)

        @pl.when(wid == _NW - 1)
        def _tail():
            base = _NBLK * _PDIM
            pltpu.sync_copy(tail_hbm, tail_v)
            transpose_block(tail_v, _DIM)
            pltpu.sync_copy(tr_v.at[pl.ds(0, _DIM), :],
                            s_hbm.at[pl.ds(base, _DIM), :])

    return detile_kernel


def _build_gather():
    mesh = plsc.VectorSubcoreMesh(core_axis_name="c", subcore_axis_name="s")

    @functools.partial(
        pl.kernel,
        out_type=jax.ShapeDtypeStruct((_NW, _NCHUNK, _CHUNK, _DIM), jnp.float32),
        mesh=mesh,
        scratch_types=[
            pltpu.VMEM((_NCHUNK, _CHUNK), jnp.int32),
            pltpu.VMEM((_NBUF, _CHUNK, _PDIM), jnp.float32),
            pltpu.SemaphoreType.DMA((_NBUF,)),
            pltpu.SemaphoreType.DMA((_NBUF,)),
        ],
        compiler_params=pltpu.CompilerParams(use_tc_tiling_on_sc=False),
    )
    def gather_kernel(table_hbm, idx_hbm, out_hbm, idx_v, rows_v, gsem, osem):
        wid = lax.axis_index("s") * 2 + lax.axis_index("c")
        pltpu.sync_copy(idx_hbm.at[wid], idx_v)

        def fire_gather(j):
            s = j % _NBUF
            pltpu.make_async_copy(
                table_hbm.at[idx_v.at[j]], rows_v.at[s], gsem.at[s]).start()

        def wait_gather(j):
            s = j % _NBUF
            pltpu.make_async_copy(
                table_hbm.at[idx_v.at[j]], rows_v.at[s], gsem.at[s]).wait()

        def out_copy(j):
            s = j % _NBUF
            return pltpu.make_async_copy(
                rows_v.at[s, :, pl.ds(0, _DIM)], out_hbm.at[wid, j],
                osem.at[s])

        fire_gather(0)
        for j in range(_NCHUNK):
            wait_gather(j)
            nxt = j + 1
            if nxt < _NCHUNK:
                if nxt >= _NBUF:
                    out_copy(nxt - _NBUF).wait()
                fire_gather(nxt)
            out_copy(j).start()
        out_copy(_NCHUNK - 2).wait()
        out_copy(_NCHUNK - 1).wait()

    return gather_kernel


_detile = _build_detile()
_gather = _build_gather()


def kernel(inputs, embeddings):
    tt = jnp.transpose(embeddings)
    tail = lax.slice(tt, (0, _NBLK * _PDIM), (_DIM, _VOCAB))
    table = _detile(tt, tail)
    idx = inputs.astype(jnp.int32).reshape(_NW, _NCHUNK, _CHUNK)
    out = _gather(table, idx)
    return out.reshape(_BATCH, _HIST, _DIM)


# speedup vs baseline: 1.2910x; 1.2910x over previous
import functools

import jax
import jax.numpy as jnp
from jax import lax
from jax.experimental import pallas as pl
from jax.experimental.pallas import tpu as pltpu
from jax.experimental.pallas import tpu_sc as plsc

_BATCH = 4096
_HIST = 50
_DIM = 64
_PDIM = 128
_VOCAB = 1000000
_NW = 32
_LANE = 16

_SPAN = 256
_NBLK = _VOCAB // _SPAN
_BLK_BASE = _NBLK // _NW
_BLK_REM = _NBLK - _BLK_BASE * _NW

_CHUNK = 400
_ROWS_PER_W = (_BATCH * _HIST) // _NW
_NCHUNK = _ROWS_PER_W // _CHUNK
_NBUF = 2


def _build_detile():
    mesh = plsc.VectorSubcoreMesh(core_axis_name="c", subcore_axis_name="s")

    @functools.partial(
        pl.kernel,
        out_type=jax.ShapeDtypeStruct((_VOCAB, _PDIM), jnp.float32),
        mesh=mesh,
        scratch_types=[
            pltpu.VMEM((2, _DIM, _SPAN), jnp.float32),
            pltpu.VMEM((_DIM, _DIM), jnp.float32),
            pltpu.VMEM((2, _SPAN, _PDIM), jnp.float32),
            pltpu.SemaphoreType.DMA((2,)),
            pltpu.SemaphoreType.DMA((2,)),
        ],
        compiler_params=pltpu.CompilerParams(
            use_tc_tiling_on_sc=True, needs_layout_passes=False),
    )
    def detile_kernel(tt_hbm, tail_hbm, s_hbm, blk_v, tail_v, tr_v,
                      lsem, ssem):
        wid = lax.axis_index("s") * 2 + lax.axis_index("c")
        start = wid * _BLK_BASE + jnp.minimum(wid, _BLK_REM)
        cnt = _BLK_BASE + jnp.where(wid < _BLK_REM, 1, 0)
        end = start + cnt
        iota = lax.iota(jnp.int32, _LANE)

        def load(i):
            s = i % 2
            return pltpu.make_async_copy(
                tt_hbm.at[:, pl.ds(i * _SPAN, _SPAN)], blk_v.at[s],
                lsem.at[s])

        def store(i):
            s = i % 2
            return pltpu.make_async_copy(
                tr_v.at[s], s_hbm.at[pl.ds(i * _SPAN, _SPAN), :],
                ssem.at[s])

        def transpose_span(s):
            svec = jnp.full((_LANE,), s, jnp.int32)
            for c in range(_DIM):
                cvec = jnp.full((_LANE,), c, jnp.int32)
                for k in range(_SPAN // _LANE):
                    v = blk_v[s, c, pl.ds(k * _LANE, _LANE)]
                    plsc.store_scatter(
                        tr_v, [svec, iota + k * _LANE, cvec], v)

        load(start).start()

        def body(i, carry):
            s = i % 2
            load(i).wait()

            @pl.when(i + 1 < end)
            def _next_load():
                load(i + 1).start()

            @pl.when(i >= start + 2)
            def _free_slot():
                store(i - 2).wait()

            transpose_span(s)
            store(i).start()
            return carry

        lax.fori_loop(start, end, body, 0)
        store(end - 2).wait()
        store(end - 1).wait()

        @pl.when(wid == _NW - 1)
        def _tail():
            base = _NBLK * _SPAN
            pltpu.sync_copy(tail_hbm, tail_v)
            svec = jnp.full((_LANE,), 0, jnp.int32)
            for c in range(_DIM):
                cvec = jnp.full((_LANE,), c, jnp.int32)
                for k in range(_DIM // _LANE):
                    v = tail_v[c, pl.ds(k * _LANE, _LANE)]
                    plsc.store_scatter(
                        tr_v, [svec, iota + k * _LANE, cvec], v)
            pltpu.sync_copy(tr_v.at[0, pl.ds(0, _DIM), :],
                            s_hbm.at[pl.ds(base, _DIM), :])

    return detile_kernel


def _build_gather():
    mesh = plsc.VectorSubcoreMesh(core_axis_name="c", subcore_axis_name="s")

    @functools.partial(
        pl.kernel,
        out_type=jax.ShapeDtypeStruct((_NW, _NCHUNK, _CHUNK, _DIM), jnp.float32),
        mesh=mesh,
        scratch_types=[
            pltpu.VMEM((_NCHUNK, _CHUNK), jnp.int32),
            pltpu.VMEM((_NBUF, _CHUNK, _PDIM), jnp.float32),
            pltpu.SemaphoreType.DMA((_NBUF,)),
            pltpu.SemaphoreType.DMA((_NBUF,)),
        ],
        compiler_params=pltpu.CompilerParams(use_tc_tiling_on_sc=False),
    )
    def gather_kernel(table_hbm, idx_hbm, out_hbm, idx_v, rows_v, gsem, osem):
        wid = lax.axis_index("s") * 2 + lax.axis_index("c")
        pltpu.sync_copy(idx_hbm.at[wid], idx_v)

        def fire_gather(j):
            s = j % _NBUF
            pltpu.make_async_copy(
                table_hbm.at[idx_v.at[j]], rows_v.at[s], gsem.at[s]).start()

        def wait_gather(j):
            s = j % _NBUF
            pltpu.make_async_copy(
                table_hbm.at[idx_v.at[j]], rows_v.at[s], gsem.at[s]).wait()

        def out_copy(j):
            s = j % _NBUF
            return pltpu.make_async_copy(
                rows_v.at[s, :, pl.ds(0, _DIM)], out_hbm.at[wid, j],
                osem.at[s])

        fire_gather(0)
        for j in range(_NCHUNK):
            wait_gather(j)
            nxt = j + 1
            if nxt < _NCHUNK:
                if nxt >= _NBUF:
                    out_copy(nxt - _NBUF).wait()
                fire_gather(nxt)
            out_copy(j).start()
        out_copy(_NCHUNK - 2).wait()
        out_copy(_NCHUNK - 1).wait()

    return gather_kernel


_detile = _build_detile()
_gather = _build_gather()


def kernel(inputs, embeddings):
    tt = jnp.transpose(embeddings)
    tail = lax.slice(tt, (0, _NBLK * _SPAN), (_DIM, _VOCAB))
    table = _detile(tt, tail)
    idx = inputs.astype(jnp.int32).reshape(_NW, _NCHUNK, _CHUNK)
    out = _gather(table, idx)
    return out.reshape(_BATCH, _HIST, _DIM)


# speedup vs baseline: 2.0156x; 1.5613x over previous
import functools

import jax
import jax.numpy as jnp
from jax import lax
from jax.experimental import pallas as pl
from jax.experimental.pallas import tpu as pltpu
from jax.experimental.pallas import tpu_sc as plsc

_BATCH = 4096
_HIST = 50
_DIM = 64
_PDIM = 128
_VOCAB = 1000000
_NW = 32
_LANE = 16

_SPAN = 256
_NBLK = _VOCAB // _SPAN
_BLK_BASE = _NBLK // _NW
_BLK_REM = _NBLK - _BLK_BASE * _NW

_CHUNK = 400
_ROWS_PER_W = (_BATCH * _HIST) // _NW
_NCHUNK = _ROWS_PER_W // _CHUNK
_NBUF = 2


_TBLK = 1024
_TGRID = -(-_VOCAB // _TBLK)


def _build_detile():
    def detile_body(tt_ref, s_ref):
        t = jnp.transpose(tt_ref[...])
        s_ref[:, : _DIM] = t
        s_ref[:, _DIM:] = jnp.zeros((_TBLK, _PDIM - _DIM), jnp.float32)

    return pl.pallas_call(
        detile_body,
        grid=(_TGRID,),
        in_specs=[pl.BlockSpec((_DIM, _TBLK), lambda i: (0, i))],
        out_specs=pl.BlockSpec((_TBLK, _PDIM), lambda i: (i, 0)),
        out_shape=jax.ShapeDtypeStruct((_VOCAB, _PDIM), jnp.float32),
    )


def _build_gather():
    mesh = plsc.VectorSubcoreMesh(core_axis_name="c", subcore_axis_name="s")

    @functools.partial(
        pl.kernel,
        out_type=jax.ShapeDtypeStruct((_NW, _NCHUNK, _CHUNK, _DIM), jnp.float32),
        mesh=mesh,
        scratch_types=[
            pltpu.VMEM((_NCHUNK, _CHUNK), jnp.int32),
            pltpu.VMEM((_NBUF, _CHUNK, _PDIM), jnp.float32),
            pltpu.SemaphoreType.DMA((_NBUF,)),
            pltpu.SemaphoreType.DMA((_NBUF,)),
        ],
        compiler_params=pltpu.CompilerParams(use_tc_tiling_on_sc=False),
    )
    def gather_kernel(table_hbm, idx_hbm, out_hbm, idx_v, rows_v, gsem, osem):
        wid = lax.axis_index("s") * 2 + lax.axis_index("c")
        pltpu.sync_copy(idx_hbm.at[wid], idx_v)

        def fire_gather(j):
            s = j % _NBUF
            pltpu.make_async_copy(
                table_hbm.at[idx_v.at[j]], rows_v.at[s], gsem.at[s]).start()

        def wait_gather(j):
            s = j % _NBUF
            pltpu.make_async_copy(
                table_hbm.at[idx_v.at[j]], rows_v.at[s], gsem.at[s]).wait()

        def out_copy(j):
            s = j % _NBUF
            return pltpu.make_async_copy(
                rows_v.at[s, :, pl.ds(0, _DIM)], out_hbm.at[wid, j],
                osem.at[s])

        fire_gather(0)
        for j in range(_NCHUNK):
            wait_gather(j)
            nxt = j + 1
            if nxt < _NCHUNK:
                if nxt >= _NBUF:
                    out_copy(nxt - _NBUF).wait()
                fire_gather(nxt)
            out_copy(j).start()
        out_copy(_NCHUNK - 2).wait()
        out_copy(_NCHUNK - 1).wait()

    return gather_kernel


_detile = _build_detile()
_gather = _build_gather()


def kernel(inputs, embeddings):
    tt = jnp.transpose(embeddings)
    table = _detile(tt)
    idx = inputs.astype(jnp.int32).reshape(_NW, _NCHUNK, _CHUNK)
    out = _gather(table, idx)
    return out.reshape(_BATCH, _HIST, _DIM)


# speedup vs baseline: 3.3758x; 1.6748x over previous
import functools

import jax
import jax.numpy as jnp
from jax import lax
from jax.experimental import pallas as pl
from jax.experimental.pallas import tpu as pltpu
from jax.experimental.pallas import tpu_sc as plsc

_BATCH = 4096
_HIST = 50
_DIM = 64
_PDIM = 128
_VOCAB = 1000000
_NW = 32
_LANE = 16

_SPAN = 256
_NBLK = _VOCAB // _SPAN
_BLK_BASE = _NBLK // _NW
_BLK_REM = _NBLK - _BLK_BASE * _NW

_CHUNK = 400
_ROWS_PER_W = (_BATCH * _HIST) // _NW
_NCHUNK = _ROWS_PER_W // _CHUNK
_NBUF = 2


_TBLK = 4096
_TGRID = -(-_VOCAB // _TBLK)


def _build_detile():
    def detile_body(tt_ref, s_ref):
        s_ref[:, : _DIM] = jnp.transpose(tt_ref[...])

    return pl.pallas_call(
        detile_body,
        grid=(_TGRID,),
        in_specs=[pl.BlockSpec((_DIM, _TBLK), lambda i: (0, i))],
        out_specs=pl.BlockSpec((_TBLK, _PDIM), lambda i: (i, 0)),
        out_shape=jax.ShapeDtypeStruct((_VOCAB, _PDIM), jnp.float32),
    )


def _build_gather():
    mesh = plsc.VectorSubcoreMesh(core_axis_name="c", subcore_axis_name="s")

    @functools.partial(
        pl.kernel,
        out_type=jax.ShapeDtypeStruct((_NW, _NCHUNK, _CHUNK, _DIM), jnp.float32),
        mesh=mesh,
        scratch_types=[
            pltpu.VMEM((_NCHUNK, _CHUNK), jnp.int32),
            pltpu.VMEM((_NBUF, _CHUNK, _PDIM), jnp.float32),
            pltpu.SemaphoreType.DMA((_NBUF,)),
            pltpu.SemaphoreType.DMA((_NBUF,)),
        ],
        compiler_params=pltpu.CompilerParams(use_tc_tiling_on_sc=False),
    )
    def gather_kernel(table_hbm, idx_hbm, out_hbm, idx_v, rows_v, gsem, osem):
        wid = lax.axis_index("s") * 2 + lax.axis_index("c")
        pltpu.sync_copy(idx_hbm.at[wid], idx_v)

        def fire_gather(j):
            s = j % _NBUF
            pltpu.make_async_copy(
                table_hbm.at[idx_v.at[j]], rows_v.at[s], gsem.at[s]).start()

        def wait_gather(j):
            s = j % _NBUF
            pltpu.make_async_copy(
                table_hbm.at[idx_v.at[j]], rows_v.at[s], gsem.at[s]).wait()

        def out_copy(j):
            s = j % _NBUF
            return pltpu.make_async_copy(
                rows_v.at[s, :, pl.ds(0, _DIM)], out_hbm.at[wid, j],
                osem.at[s])

        fire_gather(0)
        for j in range(_NCHUNK):
            wait_gather(j)
            nxt = j + 1
            if nxt < _NCHUNK:
                if nxt >= _NBUF:
                    out_copy(nxt - _NBUF).wait()
                fire_gather(nxt)
            out_copy(j).start()
        out_copy(_NCHUNK - 2).wait()
        out_copy(_NCHUNK - 1).wait()

    return gather_kernel


_detile = _build_detile()
_gather = _build_gather()


def kernel(inputs, embeddings):
    tt = jnp.transpose(embeddings)
    table = _detile(tt)
    idx = inputs.astype(jnp.int32).reshape(_NW, _NCHUNK, _CHUNK)
    out = _gather(table, idx)
    return out.reshape(_BATCH, _HIST, _DIM)


# speedup vs baseline: 3.8988x; 1.1549x over previous
import functools

import jax
import jax.numpy as jnp
from jax import lax
from jax.experimental import pallas as pl
from jax.experimental.pallas import tpu as pltpu
from jax.experimental.pallas import tpu_sc as plsc

_BATCH = 4096
_HIST = 50
_DIM = 64
_PDIM = 128
_VOCAB = 1000000
_NW = 32
_LANE = 16

_SPAN = 256
_NBLK = _VOCAB // _SPAN
_BLK_BASE = _NBLK // _NW
_BLK_REM = _NBLK - _BLK_BASE * _NW

_CHUNK = 400
_ROWS_PER_W = (_BATCH * _HIST) // _NW
_NCHUNK = _ROWS_PER_W // _CHUNK
_NBUF = 2


_TBLK = 8192
_TGRID = -(-_VOCAB // _TBLK)


def _build_detile():
    def detile_body(tt_ref, s_ref):
        s_ref[:, : _DIM] = jnp.transpose(tt_ref[...])

    return pl.pallas_call(
        detile_body,
        grid=(_TGRID,),
        in_specs=[pl.BlockSpec((_DIM, _TBLK), lambda i: (0, i))],
        out_specs=pl.BlockSpec((_TBLK, _PDIM), lambda i: (i, 0)),
        out_shape=jax.ShapeDtypeStruct((_VOCAB, _PDIM), jnp.float32),
    )


def _build_gather():
    mesh = plsc.VectorSubcoreMesh(core_axis_name="c", subcore_axis_name="s")

    @functools.partial(
        pl.kernel,
        out_type=jax.ShapeDtypeStruct((_NW, _NCHUNK, _CHUNK, _DIM), jnp.float32),
        mesh=mesh,
        scratch_types=[
            pltpu.VMEM((_NCHUNK, _CHUNK), jnp.int32),
            pltpu.VMEM((_NBUF, _CHUNK, _PDIM), jnp.float32),
            pltpu.SemaphoreType.DMA((_NBUF,)),
            pltpu.SemaphoreType.DMA((_NBUF,)),
        ],
        compiler_params=pltpu.CompilerParams(use_tc_tiling_on_sc=False),
    )
    def gather_kernel(table_hbm, idx_hbm, out_hbm, idx_v, rows_v, gsem, osem):
        wid = lax.axis_index("s") * 2 + lax.axis_index("c")
        pltpu.sync_copy(idx_hbm.at[wid], idx_v)

        def fire_gather(j):
            s = j % _NBUF
            pltpu.make_async_copy(
                table_hbm.at[idx_v.at[j]], rows_v.at[s], gsem.at[s]).start()

        def wait_gather(j):
            s = j % _NBUF
            pltpu.make_async_copy(
                table_hbm.at[idx_v.at[j]], rows_v.at[s], gsem.at[s]).wait()

        def out_copy(j):
            s = j % _NBUF
            return pltpu.make_async_copy(
                rows_v.at[s, :, pl.ds(0, _DIM)], out_hbm.at[wid, j],
                osem.at[s])

        fire_gather(0)
        for j in range(_NCHUNK):
            wait_gather(j)
            nxt = j + 1
            if nxt < _NCHUNK:
                if nxt >= _NBUF:
                    out_copy(nxt - _NBUF).wait()
                fire_gather(nxt)
            out_copy(j).start()
        out_copy(_NCHUNK - 2).wait()
        out_copy(_NCHUNK - 1).wait()

    return gather_kernel


_detile = _build_detile()
_gather = _build_gather()


def kernel(inputs, embeddings):
    tt = jnp.transpose(embeddings)
    table = _detile(tt)
    idx = inputs.astype(jnp.int32).reshape(_NW, _NCHUNK, _CHUNK)
    out = _gather(table, idx)
    return out.reshape(_BATCH, _HIST, _DIM)


# speedup vs baseline: 4.0715x; 1.0443x over previous
import functools

import jax
import jax.numpy as jnp
from jax import lax
from jax.experimental import pallas as pl
from jax.experimental.pallas import tpu as pltpu
from jax.experimental.pallas import tpu_sc as plsc

_BATCH = 4096
_HIST = 50
_DIM = 64
_PDIM = 128
_VOCAB = 1000000
_NW = 32
_LANE = 16

_SPAN = 256
_NBLK = _VOCAB // _SPAN
_BLK_BASE = _NBLK // _NW
_BLK_REM = _NBLK - _BLK_BASE * _NW

_CHUNK = 400
_ROWS_PER_W = (_BATCH * _HIST) // _NW
_NCHUNK = _ROWS_PER_W // _CHUNK
_NBUF = 2


_TBLK = 16384
_TGRID = -(-_VOCAB // _TBLK)


def _build_detile():
    def detile_body(tt_ref, s_ref):
        s_ref[:, : _DIM] = jnp.transpose(tt_ref[...])

    return pl.pallas_call(
        detile_body,
        grid=(_TGRID,),
        in_specs=[pl.BlockSpec((_DIM, _TBLK), lambda i: (0, i))],
        out_specs=pl.BlockSpec((_TBLK, _PDIM), lambda i: (i, 0)),
        out_shape=jax.ShapeDtypeStruct((_VOCAB, _PDIM), jnp.float32),
    )


def _build_gather():
    mesh = plsc.VectorSubcoreMesh(core_axis_name="c", subcore_axis_name="s")

    @functools.partial(
        pl.kernel,
        out_type=jax.ShapeDtypeStruct((_NW, _NCHUNK, _CHUNK, _DIM), jnp.float32),
        mesh=mesh,
        scratch_types=[
            pltpu.VMEM((_NCHUNK, _CHUNK), jnp.int32),
            pltpu.VMEM((_NBUF, _CHUNK, _PDIM), jnp.float32),
            pltpu.SemaphoreType.DMA((_NBUF,)),
            pltpu.SemaphoreType.DMA((_NBUF,)),
        ],
        compiler_params=pltpu.CompilerParams(use_tc_tiling_on_sc=False),
    )
    def gather_kernel(table_hbm, idx_hbm, out_hbm, idx_v, rows_v, gsem, osem):
        wid = lax.axis_index("s") * 2 + lax.axis_index("c")
        pltpu.sync_copy(idx_hbm.at[wid], idx_v)

        def fire_gather(j):
            s = j % _NBUF
            pltpu.make_async_copy(
                table_hbm.at[idx_v.at[j]], rows_v.at[s], gsem.at[s]).start()

        def wait_gather(j):
            s = j % _NBUF
            pltpu.make_async_copy(
                table_hbm.at[idx_v.at[j]], rows_v.at[s], gsem.at[s]).wait()

        def out_copy(j):
            s = j % _NBUF
            return pltpu.make_async_copy(
                rows_v.at[s, :, pl.ds(0, _DIM)], out_hbm.at[wid, j],
                osem.at[s])

        fire_gather(0)
        for j in range(_NCHUNK):
            wait_gather(j)
            nxt = j + 1
            if nxt < _NCHUNK:
                if nxt >= _NBUF:
                    out_copy(nxt - _NBUF).wait()
                fire_gather(nxt)
            out_copy(j).start()
        out_copy(_NCHUNK - 2).wait()
        out_copy(_NCHUNK - 1).wait()

    return gather_kernel


_detile = _build_detile()
_gather = _build_gather()


def kernel(inputs, embeddings):
    tt = jnp.transpose(embeddings)
    table = _detile(tt)
    idx = inputs.astype(jnp.int32).reshape(_NW, _NCHUNK, _CHUNK)
    out = _gather(table, idx)
    return out.reshape(_BATCH, _HIST, _DIM)


# speedup vs baseline: 4.1257x; 1.0133x over previous
import functools

import jax
import jax.numpy as jnp
from jax import lax
from jax.experimental import pallas as pl
from jax.experimental.pallas import tpu as pltpu
from jax.experimental.pallas import tpu_sc as plsc

_BATCH = 4096
_HIST = 50
_DIM = 64
_PDIM = 128
_VOCAB = 1000000
_NW = 32
_LANE = 16

_SPAN = 256
_NBLK = _VOCAB // _SPAN
_BLK_BASE = _NBLK // _NW
_BLK_REM = _NBLK - _BLK_BASE * _NW

_CHUNK = 400
_ROWS_PER_W = (_BATCH * _HIST) // _NW
_NCHUNK = _ROWS_PER_W // _CHUNK
_NBUF = 2


_TBLK = 32768
_TGRID = -(-_VOCAB // _TBLK)


def _build_detile():
    def detile_body(tt_ref, s_ref):
        s_ref[:, : _DIM] = jnp.transpose(tt_ref[...])

    return pl.pallas_call(
        detile_body,
        grid=(_TGRID,),
        in_specs=[pl.BlockSpec((_DIM, _TBLK), lambda i: (0, i))],
        out_specs=pl.BlockSpec((_TBLK, _PDIM), lambda i: (i, 0)),
        out_shape=jax.ShapeDtypeStruct((_VOCAB, _PDIM), jnp.float32),
    )


def _build_gather():
    mesh = plsc.VectorSubcoreMesh(core_axis_name="c", subcore_axis_name="s")

    @functools.partial(
        pl.kernel,
        out_type=jax.ShapeDtypeStruct((_NW, _NCHUNK, _CHUNK, _DIM), jnp.float32),
        mesh=mesh,
        scratch_types=[
            pltpu.VMEM((_NCHUNK, _CHUNK), jnp.int32),
            pltpu.VMEM((_NBUF, _CHUNK, _PDIM), jnp.float32),
            pltpu.SemaphoreType.DMA((_NBUF,)),
            pltpu.SemaphoreType.DMA((_NBUF,)),
        ],
        compiler_params=pltpu.CompilerParams(use_tc_tiling_on_sc=False),
    )
    def gather_kernel(table_hbm, idx_hbm, out_hbm, idx_v, rows_v, gsem, osem):
        wid = lax.axis_index("s") * 2 + lax.axis_index("c")
        pltpu.sync_copy(idx_hbm.at[wid], idx_v)

        def fire_gather(j):
            s = j % _NBUF
            pltpu.make_async_copy(
                table_hbm.at[idx_v.at[j]], rows_v.at[s], gsem.at[s]).start()

        def wait_gather(j):
            s = j % _NBUF
            pltpu.make_async_copy(
                table_hbm.at[idx_v.at[j]], rows_v.at[s], gsem.at[s]).wait()

        def out_copy(j):
            s = j % _NBUF
            return pltpu.make_async_copy(
                rows_v.at[s, :, pl.ds(0, _DIM)], out_hbm.at[wid, j],
                osem.at[s])

        fire_gather(0)
        for j in range(_NCHUNK):
            wait_gather(j)
            nxt = j + 1
            if nxt < _NCHUNK:
                if nxt >= _NBUF:
                    out_copy(nxt - _NBUF).wait()
                fire_gather(nxt)
            out_copy(j).start()
        out_copy(_NCHUNK - 2).wait()
        out_copy(_NCHUNK - 1).wait()

    return gather_kernel


_detile = _build_detile()
_gather = _build_gather()


def kernel(inputs, embeddings):
    tt = jnp.transpose(embeddings)
    table = _detile(tt)
    idx = inputs.astype(jnp.int32).reshape(_NW, _NCHUNK, _CHUNK)
    out = _gather(table, idx)
    return out.reshape(_BATCH, _HIST, _DIM)


# speedup vs baseline: 4.2012x; 1.0183x over previous
import functools

import jax
import jax.numpy as jnp
from jax import lax
from jax.experimental import pallas as pl
from jax.experimental.pallas import tpu as pltpu
from jax.experimental.pallas import tpu_sc as plsc

_BATCH = 4096
_HIST = 50
_DIM = 64
_PDIM = 128
_VOCAB = 1000000
_NW = 32
_LANE = 16

_SPAN = 256
_NBLK = _VOCAB // _SPAN
_BLK_BASE = _NBLK // _NW
_BLK_REM = _NBLK - _BLK_BASE * _NW

_CHUNK = 320
_ROWS_PER_W = (_BATCH * _HIST) // _NW
_NCHUNK = _ROWS_PER_W // _CHUNK
_NBUF = 3


_TBLK = 32768
_TGRID = -(-_VOCAB // _TBLK)


def _build_detile():
    def detile_body(tt_ref, s_ref):
        s_ref[:, : _DIM] = jnp.transpose(tt_ref[...])

    return pl.pallas_call(
        detile_body,
        grid=(_TGRID,),
        in_specs=[pl.BlockSpec((_DIM, _TBLK), lambda i: (0, i))],
        out_specs=pl.BlockSpec((_TBLK, _PDIM), lambda i: (i, 0)),
        out_shape=jax.ShapeDtypeStruct((_VOCAB, _PDIM), jnp.float32),
    )


def _build_gather():
    mesh = plsc.VectorSubcoreMesh(core_axis_name="c", subcore_axis_name="s")

    @functools.partial(
        pl.kernel,
        out_type=jax.ShapeDtypeStruct((_NW, _NCHUNK, _CHUNK, _DIM), jnp.float32),
        mesh=mesh,
        scratch_types=[
            pltpu.VMEM((_NCHUNK, _CHUNK), jnp.int32),
            pltpu.VMEM((_NBUF, _CHUNK, _PDIM), jnp.float32),
            pltpu.SemaphoreType.DMA((_NBUF,)),
            pltpu.SemaphoreType.DMA((_NBUF,)),
        ],
        compiler_params=pltpu.CompilerParams(use_tc_tiling_on_sc=False),
    )
    def gather_kernel(table_hbm, idx_hbm, out_hbm, idx_v, rows_v, gsem, osem):
        wid = lax.axis_index("s") * 2 + lax.axis_index("c")
        pltpu.sync_copy(idx_hbm.at[wid], idx_v)

        def fire_gather(j):
            s = j % _NBUF
            pltpu.make_async_copy(
                table_hbm.at[idx_v.at[j]], rows_v.at[s], gsem.at[s]).start()

        def wait_gather(j):
            s = j % _NBUF
            pltpu.make_async_copy(
                table_hbm.at[idx_v.at[j]], rows_v.at[s], gsem.at[s]).wait()

        def out_copy(j):
            s = j % _NBUF
            return pltpu.make_async_copy(
                rows_v.at[s, :, pl.ds(0, _DIM)], out_hbm.at[wid, j],
                osem.at[s])

        fire_gather(0)
        fire_gather(1)
        for j in range(_NCHUNK):
            wait_gather(j)
            nxt = j + 2
            if nxt < _NCHUNK:
                if nxt >= _NBUF:
                    out_copy(nxt - _NBUF).wait()
                fire_gather(nxt)
            out_copy(j).start()
        for j in range(_NCHUNK - _NBUF, _NCHUNK):
            out_copy(j).wait()

    return gather_kernel


_detile = _build_detile()
_gather = _build_gather()


def kernel(inputs, embeddings):
    tt = jnp.transpose(embeddings)
    table = _detile(tt)
    idx = inputs.astype(jnp.int32).reshape(_NW, _NCHUNK, _CHUNK)
    out = _gather(table, idx)
    return out.reshape(_BATCH, _HIST, _DIM)
